# baseline (speedup 1.0000x reference)
import functools

import jax
import jax.numpy as jnp
from jax import lax
from jax.experimental import pallas as pl
from jax.experimental.pallas import tpu as pltpu
from jax.experimental.pallas import tpu_sc as plsc

NC = 2
NS = 16
NW = NC * NS

G = 128
CH = 512
NB = 2
K = CH // G

PBLK = 2000


def _prep_body(e_ref, w_ref, b_ref, o_ref):
    i = pl.program_id(0)
    nvalid = pl.num_programs(0) - 1
    y = lax.dot_general(
        e_ref[...], w_ref[...], (((1,), (1,)), ((), ())),
        preferred_element_type=jnp.float32,
    )
    y = (y + b_ref[...]).astype(jnp.bfloat16)
    o_ref[...] = jnp.where(i < nvalid, y, jnp.zeros_like(y))


def _tc_prepass(emb_table, W, b):
    v, hid = emb_table.shape
    nblk = v // PBLK
    return pl.pallas_call(
        _prep_body,
        grid=(nblk + 1,),
        in_specs=[
            pl.BlockSpec((PBLK, hid), lambda i: (jnp.minimum(i, 499), 0)),
            pl.BlockSpec((hid, hid), lambda i: (0, 0)),
            pl.BlockSpec((1, hid), lambda i: (0, 0)),
        ],
        out_specs=pl.BlockSpec((PBLK, hid), lambda i: (i, 0)),
        out_shape=jax.ShapeDtypeStruct((v + PBLK, hid), jnp.bfloat16),
    )(emb_table, W, b.reshape(1, hid))


def _gather_body(ids_hbm, table_hbm, out_hbm, idx_v, rows_v, sems):
    wid = lax.axis_index("s") * NC + lax.axis_index("c")
    rw = idx_v.shape[0] * idx_v.shape[1]
    nch = rw // CH
    hw = CH // 2
    base = wid * (rw // 2)
    hid = table_hbm.shape[1]
    pltpu.sync_copy(ids_hbm.at[wid], idx_v)

    def start(c, bf):
        for h in range(2):
            for j in range(K // 2):
                pltpu.async_copy(
                    table_hbm.at[idx_v.at[c * K + h * (K // 2) + j]],
                    rows_v.at[bf, h, pl.ds(j * G, G)],
                    sems.at[bf],
                )

    def drain_and_write(c, bf):
        for h in range(2):
            pltpu.make_async_copy(
                out_hbm.at[pl.ds(0, hw), pl.ds(0, hid)],
                rows_v.at[bf, h],
                sems.at[bf],
            ).wait()
        for h in range(2):
            pltpu.sync_copy(
                rows_v.at[bf, h],
                out_hbm.at[pl.ds(base + c * hw, hw), pl.ds(h * hid, hid)],
            )

    for bf in range(NB):
        start(bf, bf)

    def group(g, _):
        for bf in range(NB):
            c = g * NB + bf
            drain_and_write(c, bf)
            start(c + NB, bf)
        return 0

    lax.fori_loop(0, nch // NB - 1, group, 0)
    for bf in range(NB):
        drain_and_write(nch - NB + bf, bf)


def _sc_gather(ids_r, table_bf16):
    n = ids_r.shape[0] * ids_r.shape[1] * ids_r.shape[2]
    hid = table_bf16.shape[1]
    rw = n // NW
    kern = functools.partial(
        pl.kernel,
        out_type=jax.ShapeDtypeStruct((n // 2, 2 * hid), jnp.bfloat16),
        mesh=plsc.VectorSubcoreMesh(core_axis_name="c", subcore_axis_name="s"),
        scratch_types=[
            pltpu.VMEM((rw // G, G), jnp.int32),
            pltpu.VMEM((NB, 2, CH // 2, hid), jnp.bfloat16),
            pltpu.SemaphoreType.DMA((NB,)),
        ],
        compiler_params=pltpu.CompilerParams(use_tc_tiling_on_sc=False),
    )(_gather_body)
    return kern(ids_r, table_bf16)


def _fin_body(x_ref, o_ref):
    o_ref[...] = x_ref[...].astype(jnp.float32)


def _tc_finish(rows_w, blk=2048):
    n2, hid2 = rows_w.shape
    return pl.pallas_call(
        _fin_body,
        grid=(n2 // blk,),
        in_specs=[pl.BlockSpec((blk, hid2), lambda i: (i, 0))],
        out_specs=pl.BlockSpec((blk, hid2), lambda i: (i, 0)),
        out_shape=jax.ShapeDtypeStruct((n2, hid2), jnp.float32),
    )(rows_w)


def kernel(token_ids, attention_mask, emb_table, W, b):
    bsz, seq = token_ids.shape
    v, hid = emb_table.shape
    n = bsz * seq
    table_bf16 = _tc_prepass(emb_table, W, b)
    ids = jnp.where(attention_mask == 0, v, token_ids).reshape(n)
    ids_r = (
        ids.reshape(NW, (n // NW) // CH, CH // 2, 2)
        .transpose(0, 1, 3, 2)
        .reshape(NW, (n // NW) // G, G)
    )
    rows_w = _sc_gather(ids_r, table_bf16)
    out_w = _tc_finish(rows_w)
    return out_w.reshape(bsz, seq, hid)


# speedup vs baseline: 1.0337x; 1.0337x over previous
import functools

import jax
import jax.numpy as jnp
from jax import lax
from jax.experimental import pallas as pl
from jax.experimental.pallas import tpu as pltpu
from jax.experimental.pallas import tpu_sc as plsc

NC = 2
NS = 16
NW = NC * NS

G = 128
CH = 512
NB = 2
K = CH // G

PBLK = 2000


def _prep_body(e_ref, w_ref, b_ref, o_ref):
    i = pl.program_id(0)
    nvalid = pl.num_programs(0) - 1
    y = lax.dot_general(
        e_ref[...], w_ref[...], (((1,), (1,)), ((), ())),
        preferred_element_type=jnp.float32,
    )
    y = (y + b_ref[...]).astype(jnp.bfloat16)
    o_ref[...] = jnp.where(i < nvalid, y, jnp.zeros_like(y))


def _tc_prepass(emb_table, W, b):
    v, hid = emb_table.shape
    nblk = v // PBLK
    return pl.pallas_call(
        _prep_body,
        grid=(nblk + 1,),
        in_specs=[
            pl.BlockSpec((PBLK, hid), lambda i: (jnp.minimum(i, 499), 0)),
            pl.BlockSpec((hid, hid), lambda i: (0, 0)),
            pl.BlockSpec((1, hid), lambda i: (0, 0)),
        ],
        out_specs=pl.BlockSpec((PBLK, hid), lambda i: (i, 0)),
        out_shape=jax.ShapeDtypeStruct((v + PBLK, hid), jnp.bfloat16),
    )(emb_table, W, b.reshape(1, hid))


def _gather_body(ids_hbm, table_hbm, out_hbm, idx_v, rows_v, sems):
    wid = lax.axis_index("s") * NC + lax.axis_index("c")
    rw = idx_v.shape[0] * idx_v.shape[1]
    nch = rw // CH
    hw = CH // 2
    base = wid * (rw // 2)
    hid = table_hbm.shape[1]
    pltpu.sync_copy(ids_hbm.at[wid], idx_v)

    def start(c, bf):
        for h in range(2):
            for j in range(K // 2):
                pltpu.async_copy(
                    table_hbm.at[idx_v.at[c * K + h * (K // 2) + j]],
                    rows_v.at[bf, h, pl.ds(j * G, G)],
                    sems.at[bf],
                )

    def drain_and_write(c, bf):
        for h in range(2):
            pltpu.make_async_copy(
                out_hbm.at[pl.ds(0, hw), pl.ds(0, hid)],
                rows_v.at[bf, h],
                sems.at[bf],
            ).wait()
        for h in range(2):
            pltpu.sync_copy(
                rows_v.at[bf, h],
                out_hbm.at[pl.ds(base + c * hw, hw), pl.ds(h * hid, hid)],
            )

    for bf in range(NB):
        start(bf, bf)

    def group(g, _):
        for bf in range(NB):
            c = g * NB + bf
            drain_and_write(c, bf)
            start(c + NB, bf)
        return 0

    lax.fori_loop(0, nch // NB - 1, group, 0)
    for bf in range(NB):
        drain_and_write(nch - NB + bf, bf)


def _sc_gather(ids_r, table_bf16):
    n = ids_r.shape[0] * ids_r.shape[1] * ids_r.shape[2]
    hid = table_bf16.shape[1]
    rw = n // NW
    kern = functools.partial(
        pl.kernel,
        out_type=jax.ShapeDtypeStruct((n // 2, 2 * hid), jnp.bfloat16),
        mesh=plsc.VectorSubcoreMesh(core_axis_name="c", subcore_axis_name="s"),
        scratch_types=[
            pltpu.VMEM((rw // G, G), jnp.int32),
            pltpu.VMEM((NB, 2, CH // 2, hid), jnp.bfloat16),
            pltpu.SemaphoreType.DMA((NB,)),
        ],
        compiler_params=pltpu.CompilerParams(use_tc_tiling_on_sc=False),
    )(_gather_body)
    return kern(ids_r, table_bf16)


def _fin_body(x_ref, o_ref):
    o_ref[...] = x_ref[...].astype(jnp.float32)


def _tc_finish(rows_w, blk=2048):
    n2, hid2 = rows_w.shape
    return pl.pallas_call(
        _fin_body,
        grid=(n2 // blk,),
        in_specs=[pl.BlockSpec((blk, hid2), lambda i: (i, 0))],
        out_specs=pl.BlockSpec((blk, hid2), lambda i: (i, 0)),
        out_shape=jax.ShapeDtypeStruct((n2, hid2), jnp.float32),
    )(rows_w)


def kernel(token_ids, attention_mask, emb_table, W, b):
    bsz, seq = token_ids.shape
    v, hid = emb_table.shape
    n = bsz * seq
    table_bf16 = _tc_prepass(emb_table, W, b)
    ids = jnp.where(attention_mask == 0, v, token_ids).reshape(n)
    ids_r = (
        ids.reshape(NW, (n // NW) // CH, CH // 2, 2)
        .transpose(0, 1, 3, 2)
        .reshape(NW, (n // NW) // G, G)
    )
    rows_w = _sc_gather(ids_r, table_bf16)
    return rows_w.astype(jnp.float32).reshape(bsz, seq, hid)
